# Initial kernel scaffold; baseline (speedup 1.0000x reference)
import functools

import jax
import jax.numpy as jnp
from jax import lax
from jax.experimental import pallas as pl
from jax.experimental.pallas import tpu as pltpu
from jax.experimental.pallas import tpu_sc as plsc

N = 10000
E = 320000
D = 128

NC = 2
NS = 16
NW = NC * NS
EPW = E // NW
C = 80
STEPS = EPW // C
RPT = N // NS

_mesh = plsc.VectorSubcoreMesh(core_axis_name="c", subcore_axis_name="s")


@functools.partial(
    pl.kernel,
    out_type=jax.ShapeDtypeStruct((NC * N, 16), jnp.float32),
    mesh=_mesh,
    scratch_types=[
        pltpu.VMEM_SHARED((N, 16), jnp.float32),
        pltpu.VMEM((STEPS, C), jnp.int32),
        pltpu.VMEM((C, 16), jnp.float32),
    ],
)
def _sc_degree(dstr_hbm, zeros_hbm, ones_hbm, out_hbm, acc, didx, ones):
    c = lax.axis_index("c")
    s = lax.axis_index("s")
    wid = s * NC + c
    pltpu.sync_copy(zeros_hbm.at[pl.ds(s * RPT, RPT)], acc.at[pl.ds(s * RPT, RPT)])
    pltpu.sync_copy(dstr_hbm.at[wid], didx)
    pltpu.sync_copy(ones_hbm, ones)
    plsc.subcore_barrier()

    def body(i, carry):
        pltpu.sync_copy(ones, acc.at[didx.at[i]], add=True)
        return carry

    lax.fori_loop(0, STEPS, body, 0)
    plsc.subcore_barrier()
    pltpu.sync_copy(acc.at[pl.ds(s * RPT, RPT)],
                    out_hbm.at[pl.ds(c * N + s * RPT, RPT)])


@functools.partial(
    pl.kernel,
    out_type=jax.ShapeDtypeStruct((NC * N, D), jnp.float32),
    mesh=_mesh,
    scratch_types=[
        pltpu.VMEM_SHARED((N, D), jnp.float32),
        pltpu.VMEM((STEPS, C), jnp.int32),
        pltpu.VMEM((STEPS, C), jnp.int32),
        pltpu.VMEM((C, D), jnp.float32),
        pltpu.SemaphoreType.DMA,
    ],
)
def _sc_scatter(hs_hbm, srcr_hbm, dstr_hbm, zeros_hbm, out_hbm,
                acc, sidx, didx, rows, sem):
    c = lax.axis_index("c")
    s = lax.axis_index("s")
    wid = s * NC + c
    pltpu.sync_copy(zeros_hbm.at[pl.ds(s * RPT, RPT)], acc.at[pl.ds(s * RPT, RPT)])
    pltpu.sync_copy(srcr_hbm.at[wid], sidx)
    pltpu.sync_copy(dstr_hbm.at[wid], didx)
    plsc.subcore_barrier()

    def body(i, carry):
        pltpu.async_copy(hs_hbm.at[sidx.at[i]], rows, sem).wait()
        pltpu.sync_copy(rows, acc.at[didx.at[i]], add=True)
        return carry

    lax.fori_loop(0, STEPS, body, 0)
    plsc.subcore_barrier()
    pltpu.sync_copy(acc.at[pl.ds(s * RPT, RPT)],
                    out_hbm.at[pl.ds(c * N + s * RPT, RPT)])


_BLK = 1000
_GRID = N // _BLK


def _dinv_blk(c0, c1):
    return lax.rsqrt(1.0 + c0[:, 0:1] + c1[:, 0:1])


def _tc_first_body(c0, c1, x, w, o):
    h = jnp.dot(x[...], w[...], preferred_element_type=jnp.float32)
    o[...] = h * _dinv_blk(c0[...], c1[...])


def _tc_mid_body(c0, c1, p0, p1, hs, b, w, o):
    dinv = _dinv_blk(c0[...], c1[...])
    t = (p0[...] + p1[...] + hs[...]) * dinv + b[...]
    t = jnp.maximum(t, 0.0)
    o[...] = jnp.dot(t, w[...], preferred_element_type=jnp.float32) * dinv


def _tc_last_body(c0, c1, p0, p1, hs, b, o):
    dinv = _dinv_blk(c0[...], c1[...])
    o[...] = (p0[...] + p1[...] + hs[...]) * dinv + b[...]


_cnt_spec = pl.BlockSpec((_BLK, 16), lambda i: (i, 0))
_row_spec = pl.BlockSpec((_BLK, D), lambda i: (i, 0))
_w_spec = pl.BlockSpec((D, D), lambda i: (0, 0))
_b_spec = pl.BlockSpec((1, D), lambda i: (0, 0))
_out_sds = jax.ShapeDtypeStruct((N, D), jnp.float32)

_tc_first = pl.pallas_call(
    _tc_first_body, grid=(_GRID,),
    in_specs=[_cnt_spec, _cnt_spec, _row_spec, _w_spec],
    out_specs=_row_spec, out_shape=_out_sds)

_tc_mid = pl.pallas_call(
    _tc_mid_body, grid=(_GRID,),
    in_specs=[_cnt_spec, _cnt_spec, _row_spec, _row_spec, _row_spec,
              _b_spec, _w_spec],
    out_specs=_row_spec, out_shape=_out_sds)

_tc_last = pl.pallas_call(
    _tc_last_body, grid=(_GRID,),
    in_specs=[_cnt_spec, _cnt_spec, _row_spec, _row_spec, _row_spec, _b_spec],
    out_specs=_row_spec, out_shape=_out_sds)


def kernel(x, edge_index, W1, b1, W2, b2):
    src = edge_index[0].astype(jnp.int32)
    dst = edge_index[1].astype(jnp.int32)
    srcr = src.reshape(NW, STEPS, C)
    dstr = dst.reshape(NW, STEPS, C)

    zeros16 = jnp.zeros((N, 16), jnp.float32)
    ones16 = jnp.ones((C, 16), jnp.float32)
    zerosD = jnp.zeros((N, D), jnp.float32)

    cnt = _sc_degree(dstr, zeros16, ones16)
    c0, c1 = cnt[:N], cnt[N:]

    hs1 = _tc_first(c0, c1, x, W1)
    p1 = _sc_scatter(hs1, srcr, dstr, zerosD)
    hs2 = _tc_mid(c0, c1, p1[:N], p1[N:], hs1, b1.reshape(1, D), W2)
    p2 = _sc_scatter(hs2, srcr, dstr, zerosD)
    out = _tc_last(c0, c1, p2[:N], p2[N:], hs2, b2.reshape(1, D))
    return out


# speedup vs baseline: 13.9971x; 13.9971x over previous
import functools

import jax
import jax.numpy as jnp
from jax import lax
from jax.experimental import pallas as pl
from jax.experimental.pallas import tpu as pltpu
from jax.experimental.pallas import tpu_sc as plsc

N = 10000
E = 320000
D = 128
H = D // 2

NC = 2
NS = 16
NW = NC * NS
C = 80

EPW = E // NW
DSTEPS = EPW // C
EPT = E // NS
SSTEPS = EPT // C

NP = 10240
RPT = NP // NS

_mesh = plsc.VectorSubcoreMesh(core_axis_name="c", subcore_axis_name="s")
_sc_params = pltpu.CompilerParams(use_tc_tiling_on_sc=False)


@functools.partial(
    pl.kernel,
    out_type=jax.ShapeDtypeStruct((NC * NP, 16), jnp.float32),
    mesh=_mesh,
    scratch_types=[
        pltpu.VMEM_SHARED((NP, 16), jnp.float32),
        pltpu.VMEM((DSTEPS, C), jnp.int32),
        pltpu.VMEM((C, 16), jnp.float32),
    ],
    compiler_params=_sc_params,
)
def _sc_degree(dstr_hbm, zeros_hbm, out_hbm, acc, didx, ones):
    c = lax.axis_index("c")
    s = lax.axis_index("s")
    wid = s * NC + c
    pltpu.sync_copy(zeros_hbm.at[pl.ds(s * RPT, RPT)], acc.at[pl.ds(s * RPT, RPT)])
    pltpu.sync_copy(dstr_hbm.at[wid], didx)

    def initb(i, carry):
        ones[i, :] = jnp.full((16,), 1.0, jnp.float32)
        return carry

    lax.fori_loop(0, C, initb, 0)
    plsc.subcore_barrier()

    def body(i, carry):
        pltpu.sync_copy(ones, acc.at[didx.at[i]], add=True)
        return carry

    lax.fori_loop(0, DSTEPS, body, 0)
    plsc.subcore_barrier()
    pltpu.sync_copy(acc.at[pl.ds(s * RPT, RPT)],
                    out_hbm.at[pl.ds(c * NP + s * RPT, RPT)])


@functools.partial(
    pl.kernel,
    out_type=jax.ShapeDtypeStruct((NC * NP, H), jnp.float32),
    mesh=_mesh,
    scratch_types=[
        pltpu.VMEM_SHARED((NP, H), jnp.float32),
        pltpu.VMEM((SSTEPS, C), jnp.int32),
        pltpu.VMEM((SSTEPS, C), jnp.int32),
        pltpu.VMEM((C, H), jnp.float32),
        pltpu.SemaphoreType.DMA,
    ],
    compiler_params=_sc_params,
)
def _sc_scatter(hsl_hbm, hsr_hbm, srcr_hbm, dstr_hbm, zeros_hbm, out_hbm,
                acc, sidx, didx, rows, sem):
    c = lax.axis_index("c")
    s = lax.axis_index("s")
    pltpu.sync_copy(zeros_hbm.at[pl.ds(s * RPT, RPT)], acc.at[pl.ds(s * RPT, RPT)])
    pltpu.sync_copy(srcr_hbm.at[s], sidx)
    pltpu.sync_copy(dstr_hbm.at[s], didx)
    plsc.subcore_barrier()

    def body(i, carry):
        @pl.when(c == 0)
        def _():
            pltpu.async_copy(hsl_hbm.at[sidx.at[i]], rows, sem).wait()

        @pl.when(c == 1)
        def _():
            pltpu.async_copy(hsr_hbm.at[sidx.at[i]], rows, sem).wait()

        pltpu.sync_copy(rows, acc.at[didx.at[i]], add=True)
        return carry

    lax.fori_loop(0, SSTEPS, body, 0)
    plsc.subcore_barrier()
    pltpu.sync_copy(acc.at[pl.ds(s * RPT, RPT)],
                    out_hbm.at[pl.ds(c * NP + s * RPT, RPT)])


_BLK = 1000
_GRID = N // _BLK


def _dinv_blk(c0, c1):
    return lax.rsqrt(1.0 + c0[:, 0:1] + c1[:, 0:1])


def _tc_first_body(c0, c1, x, w, ol, orr):
    h = jnp.dot(x[...], w[...], preferred_element_type=jnp.float32)
    hs = h * _dinv_blk(c0[...], c1[...])
    ol[...] = hs[:, :H]
    orr[...] = hs[:, H:]


def _tc_mid_body(c0, c1, p0, p1, hsl, hsr, b, w, ol, orr):
    dinv = _dinv_blk(c0[...], c1[...])
    agg = jnp.concatenate([p0[...] + hsl[...], p1[...] + hsr[...]], axis=1)
    t = jnp.maximum(agg * dinv + b[...], 0.0)
    h2 = jnp.dot(t, w[...], preferred_element_type=jnp.float32) * dinv
    ol[...] = h2[:, :H]
    orr[...] = h2[:, H:]


def _tc_last_body(c0, c1, p0, p1, hsl, hsr, b, o):
    dinv = _dinv_blk(c0[...], c1[...])
    agg = jnp.concatenate([p0[...] + hsl[...], p1[...] + hsr[...]], axis=1)
    o[...] = agg * dinv + b[...]


_cnt_spec = pl.BlockSpec((_BLK, 16), lambda i: (i, 0))
_row_spec = pl.BlockSpec((_BLK, D), lambda i: (i, 0))
_half_spec = pl.BlockSpec((_BLK, H), lambda i: (i, 0))
_w_spec = pl.BlockSpec((D, D), lambda i: (0, 0))
_b_spec = pl.BlockSpec((1, D), lambda i: (0, 0))
_half_sds = jax.ShapeDtypeStruct((N, H), jnp.float32)

_tc_first = pl.pallas_call(
    _tc_first_body, grid=(_GRID,),
    in_specs=[_cnt_spec, _cnt_spec, _row_spec, _w_spec],
    out_specs=[_half_spec, _half_spec], out_shape=[_half_sds, _half_sds])

_tc_mid = pl.pallas_call(
    _tc_mid_body, grid=(_GRID,),
    in_specs=[_cnt_spec, _cnt_spec, _half_spec, _half_spec, _half_spec,
              _half_spec, _b_spec, _w_spec],
    out_specs=[_half_spec, _half_spec], out_shape=[_half_sds, _half_sds])

_tc_last = pl.pallas_call(
    _tc_last_body, grid=(_GRID,),
    in_specs=[_cnt_spec, _cnt_spec, _half_spec, _half_spec, _half_spec,
              _half_spec, _b_spec],
    out_specs=_row_spec, out_shape=jax.ShapeDtypeStruct((N, D), jnp.float32))


def kernel(x, edge_index, W1, b1, W2, b2):
    src = edge_index[0].astype(jnp.int32)
    dst = edge_index[1].astype(jnp.int32)
    srcr_s = src.reshape(NS, SSTEPS, C)
    dstr_s = dst.reshape(NS, SSTEPS, C)
    dstr_w = dst.reshape(NW, DSTEPS, C)

    zeros16 = jnp.zeros((NP, 16), jnp.float32)
    zerosH = jnp.zeros((NP, H), jnp.float32)

    cnt = _sc_degree(dstr_w, zeros16)
    c0, c1 = cnt[:N], cnt[NP:NP + N]

    hs1l, hs1r = _tc_first(c0, c1, x, W1)
    p = _sc_scatter(hs1l, hs1r, srcr_s, dstr_s, zerosH)
    hs2l, hs2r = _tc_mid(c0, c1, p[:N], p[NP:NP + N], hs1l, hs1r,
                         b1.reshape(1, D), W2)
    p2 = _sc_scatter(hs2l, hs2r, srcr_s, dstr_s, zerosH)
    out = _tc_last(c0, c1, p2[:N], p2[NP:NP + N], hs2l, hs2r,
                   b2.reshape(1, D))
    return out


# speedup vs baseline: 30.1513x; 2.1541x over previous
import functools

import jax
import jax.numpy as jnp
from jax import lax
from jax.experimental import pallas as pl
from jax.experimental.pallas import tpu as pltpu
from jax.experimental.pallas import tpu_sc as plsc

N = 10000
E = 320000
D = 128
H = D // 2

NC = 2
NS = 16
NW = NC * NS
C = 80

EPW = E // NW
DSTEPS = EPW // C
EPT = E // NS
SSTEPS = EPT // C

NP = 10240
RPT = NP // NS

_mesh = plsc.VectorSubcoreMesh(core_axis_name="c", subcore_axis_name="s")
_sc_params = pltpu.CompilerParams(use_tc_tiling_on_sc=False)


@functools.partial(
    pl.kernel,
    out_type=jax.ShapeDtypeStruct((NC * NP, 16), jnp.float32),
    mesh=_mesh,
    scratch_types=[
        pltpu.VMEM_SHARED((NP, 16), jnp.float32),
        pltpu.VMEM((DSTEPS, C), jnp.int32),
        pltpu.VMEM((C, 16), jnp.float32),
    ],
    compiler_params=_sc_params,
)
def _sc_degree(dstr_hbm, zeros_hbm, out_hbm, acc, didx, ones):
    c = lax.axis_index("c")
    s = lax.axis_index("s")
    wid = s * NC + c
    pltpu.sync_copy(zeros_hbm.at[pl.ds(s * RPT, RPT)], acc.at[pl.ds(s * RPT, RPT)])
    pltpu.sync_copy(dstr_hbm.at[wid], didx)

    def initb(i, carry):
        ones[i, :] = jnp.full((16,), 1.0, jnp.float32)
        return carry

    lax.fori_loop(0, C, initb, 0)
    plsc.subcore_barrier()

    def body(i, carry):
        pltpu.sync_copy(ones, acc.at[didx.at[i]], add=True)
        return carry

    lax.fori_loop(0, DSTEPS, body, 0)
    plsc.subcore_barrier()
    pltpu.sync_copy(acc.at[pl.ds(s * RPT, RPT)],
                    out_hbm.at[pl.ds(c * NP + s * RPT, RPT)])


NB = 5
LA = 3


@functools.partial(
    pl.kernel,
    out_type=jax.ShapeDtypeStruct((NC * NP, H), jnp.float32),
    mesh=_mesh,
    scratch_types=[
        pltpu.VMEM_SHARED((NP, H), jnp.float32),
        pltpu.VMEM((SSTEPS, C), jnp.int32),
        pltpu.VMEM((SSTEPS, C), jnp.int32),
        pltpu.VMEM((NB, C, H), jnp.float32),
    ] + [pltpu.SemaphoreType.DMA] * (2 * NB),
    compiler_params=_sc_params,
)
def _sc_scatter(hsl_hbm, hsr_hbm, srcr_hbm, dstr_hbm, zeros_hbm, out_hbm,
                acc, sidx, didx, rows, *sems):
    gsems = sems[:NB]
    ssems = sems[NB:]
    c = lax.axis_index("c")
    s = lax.axis_index("s")
    pltpu.sync_copy(zeros_hbm.at[pl.ds(s * RPT, RPT)], acc.at[pl.ds(s * RPT, RPT)])
    pltpu.sync_copy(srcr_hbm.at[s], sidx)
    pltpu.sync_copy(dstr_hbm.at[s], didx)
    plsc.subcore_barrier()

    def gather(i, b):
        @pl.when(c == 0)
        def _():
            pltpu.async_copy(hsl_hbm.at[sidx.at[i]], rows.at[b], gsems[b])

        @pl.when(c == 1)
        def _():
            pltpu.async_copy(hsr_hbm.at[sidx.at[i]], rows.at[b], gsems[b])

    def drain(sem, b):
        pltpu.make_async_copy(zeros_hbm.at[pl.ds(0, C)], rows.at[b], sem).wait()

    for i in range(LA):
        gather(i, i % NB)

    def outer(o, carry):
        for b_off in range(NB):
            i = o * NB + b_off
            b = b_off
            bl = (b_off + LA) % NB

            @pl.when(i + LA < SSTEPS)
            def _():
                @pl.when(i >= NB - LA)
                def _():
                    drain(ssems[bl], bl)
                gather(i + LA, bl)

            drain(gsems[b], b)
            pltpu.async_copy(rows.at[b], acc.at[didx.at[i]], ssems[b])
        return carry

    lax.fori_loop(0, SSTEPS // NB, outer, 0)
    for b in range(NB):
        drain(ssems[b], b)
    plsc.subcore_barrier()
    pltpu.sync_copy(acc.at[pl.ds(s * RPT, RPT)],
                    out_hbm.at[pl.ds(c * NP + s * RPT, RPT)])


_BLK = 1000
_GRID = N // _BLK


def _dinv_blk(c0, c1):
    return lax.rsqrt(1.0 + c0[:, 0:1] + c1[:, 0:1])


def _tc_first_body(c0, c1, x, w, ol, orr):
    h = jnp.dot(x[...], w[...], preferred_element_type=jnp.float32)
    hs = h * _dinv_blk(c0[...], c1[...])
    ol[...] = hs[:, :H]
    orr[...] = hs[:, H:]


def _tc_mid_body(c0, c1, p0, p1, hsl, hsr, b, w, ol, orr):
    dinv = _dinv_blk(c0[...], c1[...])
    agg = jnp.concatenate([p0[...] + hsl[...], p1[...] + hsr[...]], axis=1)
    t = jnp.maximum(agg * dinv + b[...], 0.0)
    h2 = jnp.dot(t, w[...], preferred_element_type=jnp.float32) * dinv
    ol[...] = h2[:, :H]
    orr[...] = h2[:, H:]


def _tc_last_body(c0, c1, p0, p1, hsl, hsr, b, o):
    dinv = _dinv_blk(c0[...], c1[...])
    agg = jnp.concatenate([p0[...] + hsl[...], p1[...] + hsr[...]], axis=1)
    o[...] = agg * dinv + b[...]


_cnt_spec = pl.BlockSpec((_BLK, 16), lambda i: (i, 0))
_row_spec = pl.BlockSpec((_BLK, D), lambda i: (i, 0))
_half_spec = pl.BlockSpec((_BLK, H), lambda i: (i, 0))
_w_spec = pl.BlockSpec((D, D), lambda i: (0, 0))
_b_spec = pl.BlockSpec((1, D), lambda i: (0, 0))
_half_sds = jax.ShapeDtypeStruct((N, H), jnp.float32)

_tc_first = pl.pallas_call(
    _tc_first_body, grid=(_GRID,),
    in_specs=[_cnt_spec, _cnt_spec, _row_spec, _w_spec],
    out_specs=[_half_spec, _half_spec], out_shape=[_half_sds, _half_sds])

_tc_mid = pl.pallas_call(
    _tc_mid_body, grid=(_GRID,),
    in_specs=[_cnt_spec, _cnt_spec, _half_spec, _half_spec, _half_spec,
              _half_spec, _b_spec, _w_spec],
    out_specs=[_half_spec, _half_spec], out_shape=[_half_sds, _half_sds])

_tc_last = pl.pallas_call(
    _tc_last_body, grid=(_GRID,),
    in_specs=[_cnt_spec, _cnt_spec, _half_spec, _half_spec, _half_spec,
              _half_spec, _b_spec],
    out_specs=_row_spec, out_shape=jax.ShapeDtypeStruct((N, D), jnp.float32))


def kernel(x, edge_index, W1, b1, W2, b2):
    src = edge_index[0].astype(jnp.int32)
    dst = edge_index[1].astype(jnp.int32)
    srcr_s = src.reshape(NS, SSTEPS, C)
    dstr_s = dst.reshape(NS, SSTEPS, C)
    dstr_w = dst.reshape(NW, DSTEPS, C)

    zeros16 = jnp.zeros((NP, 16), jnp.float32)
    zerosH = jnp.zeros((NP, H), jnp.float32)

    cnt = _sc_degree(dstr_w, zeros16)
    c0, c1 = cnt[:N], cnt[NP:NP + N]

    hs1l, hs1r = _tc_first(c0, c1, x, W1)
    p = _sc_scatter(hs1l, hs1r, srcr_s, dstr_s, zerosH)
    hs2l, hs2r = _tc_mid(c0, c1, p[:N], p[NP:NP + N], hs1l, hs1r,
                         b1.reshape(1, D), W2)
    p2 = _sc_scatter(hs2l, hs2r, srcr_s, dstr_s, zerosH)
    out = _tc_last(c0, c1, p2[:N], p2[NP:NP + N], hs2l, hs2r,
                   b2.reshape(1, D))
    return out
